# baseline (speedup 1.0000x reference)
import functools

import jax
import jax.numpy as jnp
import numpy as np
from jax.experimental import pallas as pl
from jax.experimental.pallas import tpu as pltpu

T, B, D_IN, H = 215, 2048, 36, 128
D_STATIC, N_CLASSES = 9, 2
SQRT_DM = 8.0
BN_EPS = 1e-5

BB = 1024
NB = B // BB
TT = 5
NT = T // TT
SUB = 1
SR = BB // SUB


def _fused_body(src_ref, scal_ref, encWT_ref, encb_ref, Wc_ref, oadj_ref,
                lenf_ref, inv_ref, st_ref, W1h_ref, W1s_ref, b1_ref,
                W2T_ref, b2_ref, out_ref, h_ref, c_ref, pool_ref, m8_ref):
    nb = pl.program_id(0)
    nt = pl.program_id(1)

    @pl.when(nt == 0)
    def _init():
        z = jnp.zeros((BB, H), jnp.float32)
        h_ref[...] = z
        c_ref[...] = z
        pool_ref[...] = z
        row = jax.lax.broadcasted_iota(jnp.int32, (8, H), 0)
        lane = jax.lax.broadcasted_iota(jnp.int32, (8, H), 1)
        m8_ref[...] = jnp.where((row == 0) & (lane < D_IN) & (nb == 0),
                                1.0, 0.0)

    C2 = scal_ref[0, 2]
    D2 = scal_ref[0, 3]
    SB = scal_ref[0, 4]
    lenf = lenf_ref[...]
    encWT = encWT_ref[...]
    encb = encb_ref[...]
    Wc = Wc_ref[...]
    oadj = oadj_ref[...]
    m8 = m8_ref[...]

    hs = [h_ref[s * SR:(s + 1) * SR, :] for s in range(SUB)]
    cs = [c_ref[s * SR:(s + 1) * SR, :] for s in range(SUB)]
    ps = [pool_ref[s * SR:(s + 1) * SR, :] for s in range(SUB)]
    lens = [lenf[s * SR:(s + 1) * SR, :] for s in range(SUB)]

    for tt in range(TT):
        t_glob = (nt * TT + tt).astype(jnp.float32)
        for s in range(SUB):
            y = jnp.dot(src_ref[tt, s * SR:(s + 1) * SR, :].astype(jnp.bfloat16),
                        encWT, preferred_element_type=jnp.float32) + encb
            if s == 0:
                corr = 2.0 * (jnp.sum(y[0:1, :], axis=1, keepdims=True) - SB)
                y_top = y[0:8, :] + m8 * corr
                y = jnp.concatenate([y_top, y[8:, :]], axis=0)
            g = jnp.maximum(y, 0.0)
            g = jnp.maximum(g * C2 + D2, 0.0) + oadj

            xh = jnp.concatenate([g.astype(jnp.bfloat16),
                                  hs[s].astype(jnp.bfloat16)], axis=1)
            gates = jnp.dot(xh, Wc, preferred_element_type=jnp.float32)
            si = 0.5 * jnp.tanh(gates[:, 0:H]) + 0.5
            sf = 0.5 * jnp.tanh(gates[:, H:2 * H]) + 0.5
            tg = jnp.tanh(gates[:, 2 * H:3 * H])
            so = 0.5 * jnp.tanh(gates[:, 3 * H:4 * H]) + 0.5
            cs[s] = cs[s] * sf + si * tg
            hs[s] = so * jnp.tanh(cs[s])
            ps[s] = ps[s] + jnp.where(t_glob < lens[s], hs[s], 0.0)

    for s in range(SUB):
        h_ref[s * SR:(s + 1) * SR, :] = hs[s]
        c_ref[s * SR:(s + 1) * SR, :] = cs[s]
        pool_ref[s * SR:(s + 1) * SR, :] = ps[s]

    @pl.when(nt == NT - 1)
    def _finish():
        pooled = pool_ref[...] * inv_ref[...]
        hid = jnp.dot(pooled, W1h_ref[...], preferred_element_type=jnp.float32)
        hid = hid + jnp.dot(st_ref[...], W1s_ref[...],
                            preferred_element_type=jnp.float32)
        hid = jnp.maximum(hid + b1_ref[...], 0.0)
        out_ref[...] = jnp.dot(hid, W2T_ref[...],
                               preferred_element_type=jnp.float32) + b2_ref[...]


@jax.jit
def kernel(src, static, times, lengths, enc_W, enc_b, gin_w1, gin_b1,
           gin_gamma, gin_beta, gin_w2, gin_b2, lstm_Wx, lstm_bx, lstm_Wh,
           lstm_bh, mlp_W1, mlp_b1, mlp_W2, mlp_b2):
    del times
    f32 = jnp.float32

    s = 1.0 / np.sqrt(1.0 + BN_EPS)
    A = gin_w1[0, 0] * gin_gamma[0] * s
    Bc = gin_b1[0] * gin_gamma[0] * s + gin_beta[0]
    scal = jnp.stack([A, Bc, gin_w2[0, 0], gin_b2[0],
                      D_IN * Bc, jnp.float32(0.0)]).reshape(1, 6)

    encWT = jnp.zeros((D_IN, H), f32).at[:, :D_IN].set(
        enc_W.T * (SQRT_DM * A)).astype(jnp.bfloat16)
    encb = jnp.zeros((1, H), f32).at[0, :D_IN].set(enc_b * (SQRT_DM * A) + Bc)

    k_pad = jnp.maximum(gin_b2[0], 0.0)
    oadj = jnp.zeros((1, H), f32).at[0, D_IN].set(1.0 - k_pad)

    Wc = jnp.zeros((2 * H, 4 * H), f32)
    Wc = Wc.at[:D_IN, :].set(lstm_Wx.T)
    Wc = Wc.at[D_IN, :].set(lstm_bx + lstm_bh)
    Wc = Wc.at[H:, :].set(lstm_Wh.T)
    gate_scale = jnp.concatenate([
        jnp.full((H,), 0.5, f32), jnp.full((H,), 0.5, f32),
        jnp.ones((H,), f32), jnp.full((H,), 0.5, f32)]).reshape(1, 4 * H)
    Wc = (Wc * gate_scale).astype(jnp.bfloat16)

    lenf = lengths.astype(f32).reshape(B, 1)
    inv = 1.0 / (lenf + 1.0)

    W1h = mlp_W1[:, :H].T
    W1s = mlp_W1[:, H:].T
    b1 = mlp_b1.reshape(1, H + D_STATIC)
    W2T = mlp_W2.T
    b2 = mlp_b2.reshape(1, N_CLASSES)

    full = lambda shape: pl.BlockSpec(shape, lambda b, t: tuple(0 for _ in shape))

    return pl.pallas_call(
        _fused_body,
        out_shape=jax.ShapeDtypeStruct((B, N_CLASSES), f32),
        grid=(NB, NT),
        in_specs=[
            pl.BlockSpec((TT, BB, D_IN), lambda b, t: (t, b, 0)),
            full((1, 6)),
            full((D_IN, H)),
            full((1, H)),
            full((2 * H, 4 * H)),
            full((1, H)),
            pl.BlockSpec((BB, 1), lambda b, t: (b, 0)),
            pl.BlockSpec((BB, 1), lambda b, t: (b, 0)),
            pl.BlockSpec((BB, D_STATIC), lambda b, t: (b, 0)),
            full((H, H + D_STATIC)),
            full((D_STATIC, H + D_STATIC)),
            full((1, H + D_STATIC)),
            full((H + D_STATIC, N_CLASSES)),
            full((1, N_CLASSES)),
        ],
        out_specs=pl.BlockSpec((BB, N_CLASSES), lambda b, t: (b, 0)),
        scratch_shapes=[
            pltpu.VMEM((BB, H), f32),
            pltpu.VMEM((BB, H), f32),
            pltpu.VMEM((BB, H), f32),
            pltpu.VMEM((8, H), f32),
        ],
        compiler_params=pltpu.CompilerParams(
            dimension_semantics=("parallel", "arbitrary"),
        ),
        name="lstm_gin_fused",
    )(src, scal, encWT, encb, Wc, oadj, lenf, inv, static,
      W1h, W1s, b1, W2T, b2)


# speedup vs baseline: 1.0061x; 1.0061x over previous
import functools

import jax
import jax.numpy as jnp
import numpy as np
from jax.experimental import pallas as pl
from jax.experimental.pallas import tpu as pltpu

T, B, D_IN, H = 215, 2048, 36, 128
D_STATIC, N_CLASSES = 9, 2
SQRT_DM = 8.0
BN_EPS = 1e-5

BB = 1024
NB = B // BB
TT = 5
NT = T // TT


def _fused_body(src_ref, scal_ref, encWT_ref, encb_ref, Wc_ref, oadj_ref,
                lenf_ref, inv_ref, st_ref, W1h_ref, W1s_ref, b1_ref,
                W2T_ref, b2_ref, out_ref, h_ref, c_ref, pool_ref, m8_ref,
                lenb_ref):
    nb = pl.program_id(0)
    nt = pl.program_id(1)

    @pl.when(nt == 0)
    def _init():
        z = jnp.zeros((BB, H), jnp.float32)
        h_ref[...] = z
        c_ref[...] = z
        pool_ref[...] = z
        row = jax.lax.broadcasted_iota(jnp.int32, (8, H), 0)
        lane = jax.lax.broadcasted_iota(jnp.int32, (8, H), 1)
        m8_ref[...] = jnp.where((row == 0) & (lane < D_IN) & (nb == 0),
                                1.0, 0.0)
        lenb_ref[...] = jnp.broadcast_to(lenf_ref[...], (BB, H))

    C2 = scal_ref[0, 2]
    D2 = scal_ref[0, 3]
    SB = scal_ref[0, 4]
    encWT = encWT_ref[...]
    encb = encb_ref[...]
    Wc = Wc_ref[...]
    oadj = oadj_ref[...]
    m8 = m8_ref[...]

    lenb = lenb_ref[...]
    h = h_ref[...]
    c = c_ref[...]
    pool = pool_ref[...]

    for tt in range(TT):
        t_glob = (nt * TT + tt).astype(jnp.float32)
        y = jnp.dot(src_ref[tt].astype(jnp.bfloat16), encWT,
                    preferred_element_type=jnp.float32) + encb
        corr = 2.0 * (jnp.sum(y[0:1, :], axis=1, keepdims=True) - SB)
        y_top = y[0:8, :] + m8 * corr
        y = jnp.concatenate([y_top, y[8:, :]], axis=0)
        g = jnp.maximum(y, 0.0)
        g = jnp.maximum(g * C2 + D2, 0.0) + oadj

        xh = jnp.concatenate([g.astype(jnp.bfloat16),
                              h.astype(jnp.bfloat16)], axis=1)
        g_if = jnp.dot(xh, Wc[:, 0:2 * H], preferred_element_type=jnp.float32)
        si = 0.5 * jnp.tanh(g_if[:, 0:H]) + 0.5
        sf = 0.5 * jnp.tanh(g_if[:, H:2 * H]) + 0.5
        cf = c * sf
        g_go = jnp.dot(xh, Wc[:, 2 * H:4 * H],
                       preferred_element_type=jnp.float32)
        tg = jnp.tanh(g_go[:, 0:H])
        so = 0.5 * jnp.tanh(g_go[:, H:2 * H]) + 0.5
        c = cf + si * tg
        h = so * jnp.tanh(c)
        pool = pool + jnp.where(t_glob < lenb, h, 0.0)

    h_ref[...] = h
    c_ref[...] = c
    pool_ref[...] = pool

    @pl.when(nt == NT - 1)
    def _finish():
        pooled = pool_ref[...] * inv_ref[...]
        hid = jnp.dot(pooled, W1h_ref[...], preferred_element_type=jnp.float32)
        hid = hid + jnp.dot(st_ref[...], W1s_ref[...],
                            preferred_element_type=jnp.float32)
        hid = jnp.maximum(hid + b1_ref[...], 0.0)
        out_ref[...] = jnp.dot(hid, W2T_ref[...],
                               preferred_element_type=jnp.float32) + b2_ref[...]


@jax.jit
def kernel(src, static, times, lengths, enc_W, enc_b, gin_w1, gin_b1,
           gin_gamma, gin_beta, gin_w2, gin_b2, lstm_Wx, lstm_bx, lstm_Wh,
           lstm_bh, mlp_W1, mlp_b1, mlp_W2, mlp_b2):
    del times
    f32 = jnp.float32

    s = 1.0 / np.sqrt(1.0 + BN_EPS)
    A = gin_w1[0, 0] * gin_gamma[0] * s
    Bc = gin_b1[0] * gin_gamma[0] * s + gin_beta[0]
    scal = jnp.stack([A, Bc, gin_w2[0, 0], gin_b2[0],
                      D_IN * Bc, jnp.float32(0.0)]).reshape(1, 6)

    encWT = jnp.zeros((D_IN, H), f32).at[:, :D_IN].set(
        enc_W.T * (SQRT_DM * A)).astype(jnp.bfloat16)
    encb = jnp.zeros((1, H), f32).at[0, :D_IN].set(enc_b * (SQRT_DM * A) + Bc)

    k_pad = jnp.maximum(gin_b2[0], 0.0)
    oadj = jnp.zeros((1, H), f32).at[0, D_IN].set(1.0 - k_pad)

    Wc = jnp.zeros((2 * H, 4 * H), f32)
    Wc = Wc.at[:D_IN, :].set(lstm_Wx.T)
    Wc = Wc.at[D_IN, :].set(lstm_bx + lstm_bh)
    Wc = Wc.at[H:, :].set(lstm_Wh.T)
    gate_scale = jnp.concatenate([
        jnp.full((H,), 0.5, f32), jnp.full((H,), 0.5, f32),
        jnp.ones((H,), f32), jnp.full((H,), 0.5, f32)]).reshape(1, 4 * H)
    Wc = (Wc * gate_scale).astype(jnp.bfloat16)

    lenf = lengths.astype(f32).reshape(B, 1)
    inv = 1.0 / (lenf + 1.0)

    W1h = mlp_W1[:, :H].T
    W1s = mlp_W1[:, H:].T
    b1 = mlp_b1.reshape(1, H + D_STATIC)
    W2T = mlp_W2.T
    b2 = mlp_b2.reshape(1, N_CLASSES)

    full = lambda shape: pl.BlockSpec(shape, lambda b, t: tuple(0 for _ in shape))

    return pl.pallas_call(
        _fused_body,
        out_shape=jax.ShapeDtypeStruct((B, N_CLASSES), f32),
        grid=(NB, NT),
        in_specs=[
            pl.BlockSpec((TT, BB, D_IN), lambda b, t: (t, b, 0)),
            full((1, 6)),
            full((D_IN, H)),
            full((1, H)),
            full((2 * H, 4 * H)),
            full((1, H)),
            pl.BlockSpec((BB, 1), lambda b, t: (b, 0)),
            pl.BlockSpec((BB, 1), lambda b, t: (b, 0)),
            pl.BlockSpec((BB, D_STATIC), lambda b, t: (b, 0)),
            full((H, H + D_STATIC)),
            full((D_STATIC, H + D_STATIC)),
            full((1, H + D_STATIC)),
            full((H + D_STATIC, N_CLASSES)),
            full((1, N_CLASSES)),
        ],
        out_specs=pl.BlockSpec((BB, N_CLASSES), lambda b, t: (b, 0)),
        scratch_shapes=[
            pltpu.VMEM((BB, H), f32),
            pltpu.VMEM((BB, H), f32),
            pltpu.VMEM((BB, H), f32),
            pltpu.VMEM((8, H), f32),
            pltpu.VMEM((BB, H), f32),
        ],
        compiler_params=pltpu.CompilerParams(
            dimension_semantics=("parallel", "arbitrary"),
        ),
        name="lstm_gin_fused",
    )(src, scal, encWT, encb, Wc, oadj, lenf, inv, static,
      W1h, W1s, b1, W2T, b2)


# speedup vs baseline: 1.1659x; 1.1589x over previous
import functools

import jax
import jax.numpy as jnp
import numpy as np
from jax.experimental import pallas as pl
from jax.experimental.pallas import tpu as pltpu

T, B, D_IN, H = 215, 2048, 36, 128
D_STATIC, N_CLASSES = 9, 2
SQRT_DM = 8.0
BN_EPS = 1e-5

BB = 1024
NB = B // BB
TT = 5
NT = T // TT


def _fused_body(src_ref, scal_ref, encWT_ref, encb_ref, Wc_ref, oadj_ref,
                lenf_ref, inv_ref, st_ref, W1h_ref, W1s_ref, b1_ref,
                W2T_ref, b2_ref, out_ref, h_ref, c_ref, pool_ref, m8_ref,
                lenb_ref, gbuf_ref):
    nb = pl.program_id(0)
    nt = pl.program_id(1)

    @pl.when(nt == 0)
    def _init():
        z = jnp.zeros((BB, H), jnp.float32)
        h_ref[...] = z
        c_ref[...] = z
        pool_ref[...] = z
        row = jax.lax.broadcasted_iota(jnp.int32, (8, H), 0)
        lane = jax.lax.broadcasted_iota(jnp.int32, (8, H), 1)
        m8_ref[...] = jnp.where((row == 0) & (lane < D_IN) & (nb == 0),
                                1.0, 0.0)
        lenb_ref[...] = jnp.broadcast_to(lenf_ref[...], (BB, H))

    C2 = scal_ref[0, 2]
    D2 = scal_ref[0, 3]
    SB = scal_ref[0, 4]
    encWT = encWT_ref[...]
    encb = encb_ref[...]
    Wc = Wc_ref[...]
    oadj = oadj_ref[...]
    m8 = m8_ref[...]

    for tt in range(TT):
        y = jnp.dot(src_ref[tt].astype(jnp.bfloat16), encWT,
                    preferred_element_type=jnp.float32) + encb
        corr = 2.0 * (jnp.sum(y[0:1, :], axis=1, keepdims=True) - SB)
        y_top = y[0:8, :] + m8 * corr
        y = jnp.concatenate([y_top, y[8:, :]], axis=0)
        g = jnp.maximum(y, 0.0)
        g = jnp.maximum(g * C2 + D2, 0.0) + oadj
        gbuf_ref[tt] = g.astype(jnp.bfloat16)

    lenb = lenb_ref[...]
    h2 = h_ref[...]
    c = c_ref[...]
    pool = pool_ref[...]

    for tt in range(TT):
        t_glob = (nt * TT + tt).astype(jnp.float32)
        xh = jnp.concatenate([gbuf_ref[tt], h2.astype(jnp.bfloat16)], axis=1)
        g_if = jnp.dot(xh, Wc[:, 0:2 * H], preferred_element_type=jnp.float32)
        i1 = jnp.tanh(g_if[:, 0:H]) + 1.0
        f1 = jnp.tanh(g_if[:, H:2 * H]) + 1.0
        ca = c * f1
        g_go = jnp.dot(xh, Wc[:, 2 * H:4 * H],
                       preferred_element_type=jnp.float32)
        tb = jnp.tanh(g_go[:, 0:H]) * i1
        c = 0.5 * (ca + tb)
        h2 = (jnp.tanh(g_go[:, H:2 * H]) + 1.0) * jnp.tanh(c)
        pool = pool + jnp.where(t_glob < lenb, h2, 0.0)

    h_ref[...] = h2
    c_ref[...] = c
    pool_ref[...] = pool

    @pl.when(nt == NT - 1)
    def _finish():
        pooled = pool_ref[...] * inv_ref[...]
        hid = jnp.dot(pooled, W1h_ref[...], preferred_element_type=jnp.float32)
        hid = hid + jnp.dot(st_ref[...], W1s_ref[...],
                            preferred_element_type=jnp.float32)
        hid = jnp.maximum(hid + b1_ref[...], 0.0)
        out_ref[...] = jnp.dot(hid, W2T_ref[...],
                               preferred_element_type=jnp.float32) + b2_ref[...]


@jax.jit
def kernel(src, static, times, lengths, enc_W, enc_b, gin_w1, gin_b1,
           gin_gamma, gin_beta, gin_w2, gin_b2, lstm_Wx, lstm_bx, lstm_Wh,
           lstm_bh, mlp_W1, mlp_b1, mlp_W2, mlp_b2):
    del times
    f32 = jnp.float32

    s = 1.0 / np.sqrt(1.0 + BN_EPS)
    A = gin_w1[0, 0] * gin_gamma[0] * s
    Bc = gin_b1[0] * gin_gamma[0] * s + gin_beta[0]
    scal = jnp.stack([A, Bc, gin_w2[0, 0], gin_b2[0],
                      D_IN * Bc, jnp.float32(0.0)]).reshape(1, 6)

    encWT = jnp.zeros((D_IN, H), f32).at[:, :D_IN].set(
        enc_W.T * (SQRT_DM * A)).astype(jnp.bfloat16)
    encb = jnp.zeros((1, H), f32).at[0, :D_IN].set(enc_b * (SQRT_DM * A) + Bc)

    k_pad = jnp.maximum(gin_b2[0], 0.0)
    oadj = jnp.zeros((1, H), f32).at[0, D_IN].set(1.0 - k_pad)

    Wc = jnp.zeros((2 * H, 4 * H), f32)
    Wc = Wc.at[:D_IN, :].set(lstm_Wx.T)
    Wc = Wc.at[D_IN, :].set(lstm_bx + lstm_bh)
    Wc = Wc.at[H:, :].set(lstm_Wh.T)
    gate_scale = jnp.concatenate([
        jnp.full((H,), 0.5, f32), jnp.full((H,), 0.5, f32),
        jnp.ones((H,), f32), jnp.full((H,), 0.5, f32)]).reshape(1, 4 * H)
    Wc = Wc * gate_scale
    Wc = Wc.at[H:, :].mul(0.5)
    Wc = Wc.astype(jnp.bfloat16)

    lenf = lengths.astype(f32).reshape(B, 1)
    inv = 0.5 / (lenf + 1.0)

    W1h = mlp_W1[:, :H].T
    W1s = mlp_W1[:, H:].T
    b1 = mlp_b1.reshape(1, H + D_STATIC)
    W2T = mlp_W2.T
    b2 = mlp_b2.reshape(1, N_CLASSES)

    full = lambda shape: pl.BlockSpec(shape, lambda b, t: tuple(0 for _ in shape))

    return pl.pallas_call(
        _fused_body,
        out_shape=jax.ShapeDtypeStruct((B, N_CLASSES), f32),
        grid=(NB, NT),
        in_specs=[
            pl.BlockSpec((TT, BB, D_IN), lambda b, t: (t, b, 0)),
            full((1, 6)),
            full((D_IN, H)),
            full((1, H)),
            full((2 * H, 4 * H)),
            full((1, H)),
            pl.BlockSpec((BB, 1), lambda b, t: (b, 0)),
            pl.BlockSpec((BB, 1), lambda b, t: (b, 0)),
            pl.BlockSpec((BB, D_STATIC), lambda b, t: (b, 0)),
            full((H, H + D_STATIC)),
            full((D_STATIC, H + D_STATIC)),
            full((1, H + D_STATIC)),
            full((H + D_STATIC, N_CLASSES)),
            full((1, N_CLASSES)),
        ],
        out_specs=pl.BlockSpec((BB, N_CLASSES), lambda b, t: (b, 0)),
        scratch_shapes=[
            pltpu.VMEM((BB, H), f32),
            pltpu.VMEM((BB, H), f32),
            pltpu.VMEM((BB, H), f32),
            pltpu.VMEM((8, H), f32),
            pltpu.VMEM((BB, H), f32),
            pltpu.VMEM((TT, BB, H), jnp.bfloat16),
        ],
        compiler_params=pltpu.CompilerParams(
            dimension_semantics=("parallel", "arbitrary"),
        ),
        name="lstm_gin_fused",
    )(src, scal, encWT, encb, Wc, oadj, lenf, inv, static,
      W1h, W1s, b1, W2T, b2)


# speedup vs baseline: 1.2093x; 1.0372x over previous
import functools

import jax
import jax.numpy as jnp
import numpy as np
from jax.experimental import pallas as pl
from jax.experimental.pallas import tpu as pltpu

T, B, D_IN, H = 215, 2048, 36, 128
D_STATIC, N_CLASSES = 9, 2
SQRT_DM = 8.0
BN_EPS = 1e-5

BB = 2048
NB = B // BB
TT = 5
NT = T // TT


def _fused_body(src_ref, scal_ref, encWT_ref, encb_ref, Wc_ref, oadj_ref,
                lenf_ref, inv_ref, st_ref, W1h_ref, W1s_ref, b1_ref,
                W2T_ref, b2_ref, out_ref, h_ref, c_ref, pool_ref, m8_ref,
                lenb_ref, gbuf_ref):
    nb = pl.program_id(0)
    nt = pl.program_id(1)

    @pl.when(nt == 0)
    def _init():
        z = jnp.zeros((BB, H), jnp.float32)
        h_ref[...] = z
        c_ref[...] = z
        pool_ref[...] = z
        row = jax.lax.broadcasted_iota(jnp.int32, (8, H), 0)
        lane = jax.lax.broadcasted_iota(jnp.int32, (8, H), 1)
        m8_ref[...] = jnp.where((row == 0) & (lane < D_IN) & (nb == 0),
                                1.0, 0.0)
        lenb_ref[...] = jnp.broadcast_to(lenf_ref[...], (BB, H))

    C2 = scal_ref[0, 2]
    D2 = scal_ref[0, 3]
    SB = scal_ref[0, 4]
    encWT = encWT_ref[...]
    encb = encb_ref[...]
    Wc = Wc_ref[...]
    oadj = oadj_ref[...]
    m8 = m8_ref[...]

    def enc_gin(tt):
        y = jnp.dot(src_ref[tt].astype(jnp.bfloat16), encWT,
                    preferred_element_type=jnp.float32) + encb
        corr = 2.0 * (jnp.sum(y[0:1, :], axis=1, keepdims=True) - SB)
        y_top = y[0:8, :] + m8 * corr
        y = jnp.concatenate([y_top, y[8:, :]], axis=0)
        g = jnp.maximum(y, 0.0)
        g = jnp.maximum(g * C2 + D2, 0.0) + oadj
        return g.astype(jnp.bfloat16)

    for tt in range(TT):
        gbuf_ref[tt] = enc_gin(tt)

    lenb = lenb_ref[...]
    h2 = h_ref[...]
    c = c_ref[...]
    pool = pool_ref[...]

    for tt in range(TT):
        t_glob = (nt * TT + tt).astype(jnp.float32)
        xh = jnp.concatenate([gbuf_ref[tt], h2.astype(jnp.bfloat16)], axis=1)
        gates = jnp.dot(xh, Wc, preferred_element_type=jnp.float32)
        g_if = gates[:, 0:2 * H]
        g_go = gates[:, 2 * H:4 * H]
        i1 = jnp.tanh(g_if[:, 0:H]) + 1.0
        f1 = jnp.tanh(g_if[:, H:2 * H]) + 1.0
        ca = c * f1
        tb = jnp.tanh(g_go[:, 0:H]) * i1
        c = 0.5 * (ca + tb)
        h2 = (jnp.tanh(g_go[:, H:2 * H]) + 1.0) * jnp.tanh(c)
        pool = pool + jnp.where(t_glob < lenb, h2, 0.0)

    h_ref[...] = h2
    c_ref[...] = c
    pool_ref[...] = pool

    @pl.when(nt == NT - 1)
    def _finish():
        pooled = pool_ref[...] * inv_ref[...]
        hid = jnp.dot(pooled, W1h_ref[...], preferred_element_type=jnp.float32)
        hid = hid + jnp.dot(st_ref[...], W1s_ref[...],
                            preferred_element_type=jnp.float32)
        hid = jnp.maximum(hid + b1_ref[...], 0.0)
        out_ref[...] = jnp.dot(hid, W2T_ref[...],
                               preferred_element_type=jnp.float32) + b2_ref[...]


@jax.jit
def kernel(src, static, times, lengths, enc_W, enc_b, gin_w1, gin_b1,
           gin_gamma, gin_beta, gin_w2, gin_b2, lstm_Wx, lstm_bx, lstm_Wh,
           lstm_bh, mlp_W1, mlp_b1, mlp_W2, mlp_b2):
    del times
    f32 = jnp.float32

    s = 1.0 / np.sqrt(1.0 + BN_EPS)
    A = gin_w1[0, 0] * gin_gamma[0] * s
    Bc = gin_b1[0] * gin_gamma[0] * s + gin_beta[0]
    scal = jnp.stack([A, Bc, gin_w2[0, 0], gin_b2[0],
                      D_IN * Bc, jnp.float32(0.0)]).reshape(1, 6)

    encWT = jnp.zeros((D_IN, H), f32).at[:, :D_IN].set(
        enc_W.T * (SQRT_DM * A)).astype(jnp.bfloat16)
    encb = jnp.zeros((1, H), f32).at[0, :D_IN].set(enc_b * (SQRT_DM * A) + Bc)

    k_pad = jnp.maximum(gin_b2[0], 0.0)
    oadj = jnp.zeros((1, H), f32).at[0, D_IN].set(1.0 - k_pad)

    Wc = jnp.zeros((2 * H, 4 * H), f32)
    Wc = Wc.at[:D_IN, :].set(lstm_Wx.T)
    Wc = Wc.at[D_IN, :].set(lstm_bx + lstm_bh)
    Wc = Wc.at[H:, :].set(lstm_Wh.T)
    gate_scale = jnp.concatenate([
        jnp.full((H,), 0.5, f32), jnp.full((H,), 0.5, f32),
        jnp.ones((H,), f32), jnp.full((H,), 0.5, f32)]).reshape(1, 4 * H)
    Wc = Wc * gate_scale
    Wc = Wc.at[H:, :].mul(0.5)
    Wc = Wc.astype(jnp.bfloat16)

    lenf = lengths.astype(f32).reshape(B, 1)
    inv = 0.5 / (lenf + 1.0)

    W1h = mlp_W1[:, :H].T
    W1s = mlp_W1[:, H:].T
    b1 = mlp_b1.reshape(1, H + D_STATIC)
    W2T = mlp_W2.T
    b2 = mlp_b2.reshape(1, N_CLASSES)

    full = lambda shape: pl.BlockSpec(shape, lambda b, t: tuple(0 for _ in shape))

    return pl.pallas_call(
        _fused_body,
        out_shape=jax.ShapeDtypeStruct((B, N_CLASSES), f32),
        grid=(NB, NT),
        in_specs=[
            pl.BlockSpec((TT, BB, D_IN), lambda b, t: (t, b, 0)),
            full((1, 6)),
            full((D_IN, H)),
            full((1, H)),
            full((2 * H, 4 * H)),
            full((1, H)),
            pl.BlockSpec((BB, 1), lambda b, t: (b, 0)),
            pl.BlockSpec((BB, 1), lambda b, t: (b, 0)),
            pl.BlockSpec((BB, D_STATIC), lambda b, t: (b, 0)),
            full((H, H + D_STATIC)),
            full((D_STATIC, H + D_STATIC)),
            full((1, H + D_STATIC)),
            full((H + D_STATIC, N_CLASSES)),
            full((1, N_CLASSES)),
        ],
        out_specs=pl.BlockSpec((BB, N_CLASSES), lambda b, t: (b, 0)),
        scratch_shapes=[
            pltpu.VMEM((BB, H), f32),
            pltpu.VMEM((BB, H), f32),
            pltpu.VMEM((BB, H), f32),
            pltpu.VMEM((8, H), f32),
            pltpu.VMEM((BB, H), f32),
            pltpu.VMEM((TT, BB, H), jnp.bfloat16),
        ],
        compiler_params=pltpu.CompilerParams(
            dimension_semantics=("parallel", "arbitrary"),
        ),
        name="lstm_gin_fused",
    )(src, scal, encWT, encb, Wc, oadj, lenf, inv, static,
      W1h, W1s, b1, W2T, b2)


# speedup vs baseline: 1.2147x; 1.0044x over previous
import jax
import jax.numpy as jnp
import numpy as np
from jax.experimental import pallas as pl
from jax.experimental.pallas import tpu as pltpu

T, B, D_IN, H = 215, 2048, 36, 128
D_STATIC, N_CLASSES = 9, 2
SQRT_DM = 8.0
BN_EPS = 1e-5

BB = 512
NB = B // BB
TT = 43
NT = T // TT


def _fused_body(src_ref, scal_ref, encWT_ref, encb_ref, Wc_ref, oadj_ref,
                lenf_ref, inv_ref, st_ref, W1h_ref, W1s_ref, b1_ref,
                W2T_ref, b2_ref, out_ref, h_ref, c_ref, pool_ref, m8_ref,
                lenb_ref, gbuf_ref):
    nb = pl.program_id(0)
    nt = pl.program_id(1)

    @pl.when(nt == 0)
    def _init():
        z = jnp.zeros((BB, H), jnp.float32)
        h_ref[...] = z
        c_ref[...] = z
        pool_ref[...] = z
        row = jax.lax.broadcasted_iota(jnp.int32, (8, H), 0)
        lane = jax.lax.broadcasted_iota(jnp.int32, (8, H), 1)
        m8_ref[...] = jnp.where((row == 0) & (lane < D_IN) & (nb == 0),
                                1.0, 0.0)
        lenb_ref[...] = jnp.broadcast_to(lenf_ref[...], (BB, H))

    C2 = scal_ref[0, 2]
    D2 = scal_ref[0, 3]
    SB = scal_ref[0, 4]
    encWT = encWT_ref[...]
    encb = encb_ref[...]
    Wc = Wc_ref[...]
    oadj = oadj_ref[...]
    m8 = m8_ref[...]

    def enc_gin(tt):
        y = jnp.dot(src_ref[tt].astype(jnp.bfloat16), encWT,
                    preferred_element_type=jnp.float32) + encb
        corr = 2.0 * (jnp.sum(y[0:1, :], axis=1, keepdims=True) - SB)
        y_top = y[0:8, :] + m8 * corr
        y = jnp.concatenate([y_top, y[8:, :]], axis=0)
        g = jnp.maximum(y, 0.0)
        g = jnp.maximum(g * C2 + D2, 0.0) + oadj
        return g.astype(jnp.bfloat16)

    for tt in range(TT):
        gbuf_ref[tt] = enc_gin(tt)

    lenb = lenb_ref[...]
    h2 = h_ref[...]
    c = c_ref[...]
    pool = pool_ref[...]

    for tt in range(TT):
        t_glob = (nt * TT + tt).astype(jnp.float32)
        xh = jnp.concatenate([gbuf_ref[tt], h2.astype(jnp.bfloat16)], axis=1)
        gates = jnp.dot(xh, Wc, preferred_element_type=jnp.float32)
        g_if = gates[:, 0:2 * H]
        g_go = gates[:, 2 * H:4 * H]
        i1 = jnp.tanh(g_if[:, 0:H]) + 1.0
        f1 = jnp.tanh(g_if[:, H:2 * H]) + 1.0
        ca = c * f1
        tb = jnp.tanh(g_go[:, 0:H]) * i1
        c = 0.5 * (ca + tb)
        h2 = (jnp.tanh(g_go[:, H:2 * H]) + 1.0) * jnp.tanh(c)
        pool = pool + jnp.where(t_glob < lenb, h2, 0.0)

    h_ref[...] = h2
    c_ref[...] = c
    pool_ref[...] = pool

    @pl.when(nt == NT - 1)
    def _finish():
        pooled = pool_ref[...] * inv_ref[...]
        hid = jnp.dot(pooled, W1h_ref[...], preferred_element_type=jnp.float32)
        hid = hid + jnp.dot(st_ref[...], W1s_ref[...],
                            preferred_element_type=jnp.float32)
        hid = jnp.maximum(hid + b1_ref[...], 0.0)
        out_ref[...] = jnp.dot(hid, W2T_ref[...],
                               preferred_element_type=jnp.float32) + b2_ref[...]


@jax.jit
def kernel(src, static, times, lengths, enc_W, enc_b, gin_w1, gin_b1,
           gin_gamma, gin_beta, gin_w2, gin_b2, lstm_Wx, lstm_bx, lstm_Wh,
           lstm_bh, mlp_W1, mlp_b1, mlp_W2, mlp_b2):
    del times
    f32 = jnp.float32

    s = 1.0 / np.sqrt(1.0 + BN_EPS)
    A = gin_w1[0, 0] * gin_gamma[0] * s
    Bc = gin_b1[0] * gin_gamma[0] * s + gin_beta[0]
    scal = jnp.stack([A, Bc, gin_w2[0, 0], gin_b2[0],
                      D_IN * Bc, jnp.float32(0.0)]).reshape(1, 6)

    encWT = jnp.zeros((D_IN, H), f32).at[:, :D_IN].set(
        enc_W.T * (SQRT_DM * A)).astype(jnp.bfloat16)
    encb = jnp.zeros((1, H), f32).at[0, :D_IN].set(enc_b * (SQRT_DM * A) + Bc)

    k_pad = jnp.maximum(gin_b2[0], 0.0)
    oadj = jnp.zeros((1, H), f32).at[0, D_IN].set(1.0 - k_pad)

    Wc = jnp.zeros((2 * H, 4 * H), f32)
    Wc = Wc.at[:D_IN, :].set(lstm_Wx.T)
    Wc = Wc.at[D_IN, :].set(lstm_bx + lstm_bh)
    Wc = Wc.at[H:, :].set(lstm_Wh.T)
    gate_scale = jnp.concatenate([
        jnp.full((H,), 0.5, f32), jnp.full((H,), 0.5, f32),
        jnp.ones((H,), f32), jnp.full((H,), 0.5, f32)]).reshape(1, 4 * H)
    Wc = Wc * gate_scale
    Wc = Wc.at[H:, :].mul(0.5)
    Wc = Wc.astype(jnp.bfloat16)

    lenf = lengths.astype(f32).reshape(B, 1)
    inv = 0.5 / (lenf + 1.0)

    W1h = mlp_W1[:, :H].T
    W1s = mlp_W1[:, H:].T
    b1 = mlp_b1.reshape(1, H + D_STATIC)
    W2T = mlp_W2.T
    b2 = mlp_b2.reshape(1, N_CLASSES)

    full = lambda shape: pl.BlockSpec(shape, lambda b, t: tuple(0 for _ in shape))

    return pl.pallas_call(
        _fused_body,
        out_shape=jax.ShapeDtypeStruct((B, N_CLASSES), f32),
        grid=(NB, NT),
        in_specs=[
            pl.BlockSpec((TT, BB, D_IN), lambda b, t: (t, b, 0)),
            full((1, 6)),
            full((D_IN, H)),
            full((1, H)),
            full((2 * H, 4 * H)),
            full((1, H)),
            pl.BlockSpec((BB, 1), lambda b, t: (b, 0)),
            pl.BlockSpec((BB, 1), lambda b, t: (b, 0)),
            pl.BlockSpec((BB, D_STATIC), lambda b, t: (b, 0)),
            full((H, H + D_STATIC)),
            full((D_STATIC, H + D_STATIC)),
            full((1, H + D_STATIC)),
            full((H + D_STATIC, N_CLASSES)),
            full((1, N_CLASSES)),
        ],
        out_specs=pl.BlockSpec((BB, N_CLASSES), lambda b, t: (b, 0)),
        scratch_shapes=[
            pltpu.VMEM((BB, H), f32),
            pltpu.VMEM((BB, H), f32),
            pltpu.VMEM((BB, H), f32),
            pltpu.VMEM((8, H), f32),
            pltpu.VMEM((BB, H), f32),
            pltpu.VMEM((TT, BB, H), jnp.bfloat16),
        ],
        compiler_params=pltpu.CompilerParams(
            dimension_semantics=("parallel", "arbitrary"),
        ),
        name="lstm_gin_fused",
    )(src, scal, encWT, encb, Wc, oadj, lenf, inv, static,
      W1h, W1s, b1, W2T, b2)


# speedup vs baseline: 1.2474x; 1.0269x over previous
import jax
import jax.numpy as jnp
import numpy as np
from jax.experimental import pallas as pl
from jax.experimental.pallas import tpu as pltpu

T, B, D_IN, H = 215, 2048, 36, 128
D_STATIC, N_CLASSES = 9, 2
SQRT_DM = 8.0
BN_EPS = 1e-5

BB = 512
NB = B // BB
TT = 43
NT = T // TT


def _fused_body(src_ref, scal_ref, encWT_ref, encb_ref, Wc_ref, oadj_ref,
                lenf_ref, inv_ref, st_ref, W1h_ref, W1s_ref, b1_ref,
                W2T_ref, b2_ref, out_ref, h_ref, c_ref, pool_ref, m8_ref,
                lenb_ref, gbuf_ref):
    nb = pl.program_id(0)
    nt = pl.program_id(1)

    @pl.when(nt == 0)
    def _init():
        z = jnp.zeros((BB, H), jnp.float32)
        h_ref[...] = z
        c_ref[...] = z
        pool_ref[...] = z
        row = jax.lax.broadcasted_iota(jnp.int32, (8, H), 0)
        lane = jax.lax.broadcasted_iota(jnp.int32, (8, H), 1)
        m8_ref[...] = jnp.where((row == 0) & (lane < D_IN) & (nb == 0),
                                1.0, 0.0)
        lenb_ref[...] = jnp.broadcast_to(lenf_ref[...], (BB, H))

    C2 = scal_ref[0, 2]
    D2 = scal_ref[0, 3]
    SB = scal_ref[0, 4]
    encWT = encWT_ref[...]
    encb = encb_ref[...]
    Wc = Wc_ref[...]
    oadj = oadj_ref[...]
    m8 = m8_ref[...]

    def enc_gin(tt):
        y = jnp.dot(src_ref[tt], encWT,
                    preferred_element_type=jnp.float32) + encb
        corr = 2.0 * (jnp.sum(y[0:1, :], axis=1, keepdims=True) - SB)
        y_top = y[0:8, :] + m8 * corr
        y = jnp.concatenate([y_top, y[8:, :]], axis=0)
        g = jnp.maximum(y, 0.0)
        g = jnp.maximum(g * C2 + D2, 0.0) + oadj
        return g

    for tt in range(TT):
        gbuf_ref[tt] = enc_gin(tt)

    lenb = lenb_ref[...]
    h2 = h_ref[...]
    c = c_ref[...]
    pool = pool_ref[...]

    for tt in range(TT):
        t_glob = (nt * TT + tt).astype(jnp.float32)
        xh = jnp.concatenate([gbuf_ref[tt], h2], axis=1)
        gates = jnp.dot(xh, Wc, preferred_element_type=jnp.float32)
        g_if = gates[:, 0:2 * H]
        g_go = gates[:, 2 * H:4 * H]
        i1 = jnp.tanh(g_if[:, 0:H]) + 1.0
        f1 = jnp.tanh(g_if[:, H:2 * H]) + 1.0
        ca = c * f1
        tb = jnp.tanh(g_go[:, 0:H]) * i1
        c = 0.5 * (ca + tb)
        h2 = (jnp.tanh(g_go[:, H:2 * H]) + 1.0) * jnp.tanh(c)
        pool = pool + jnp.where(t_glob < lenb, h2, 0.0)

    h_ref[...] = h2
    c_ref[...] = c
    pool_ref[...] = pool

    @pl.when(nt == NT - 1)
    def _finish():
        pooled = pool_ref[...] * inv_ref[...]
        hid = jnp.dot(pooled, W1h_ref[...], preferred_element_type=jnp.float32)
        hid = hid + jnp.dot(st_ref[...], W1s_ref[...],
                            preferred_element_type=jnp.float32)
        hid = jnp.maximum(hid + b1_ref[...], 0.0)
        out_ref[...] = jnp.dot(hid, W2T_ref[...],
                               preferred_element_type=jnp.float32) + b2_ref[...]


@jax.jit
def kernel(src, static, times, lengths, enc_W, enc_b, gin_w1, gin_b1,
           gin_gamma, gin_beta, gin_w2, gin_b2, lstm_Wx, lstm_bx, lstm_Wh,
           lstm_bh, mlp_W1, mlp_b1, mlp_W2, mlp_b2):
    del times
    f32 = jnp.float32

    s = 1.0 / np.sqrt(1.0 + BN_EPS)
    A = gin_w1[0, 0] * gin_gamma[0] * s
    Bc = gin_b1[0] * gin_gamma[0] * s + gin_beta[0]
    scal = jnp.stack([A, Bc, gin_w2[0, 0], gin_b2[0],
                      D_IN * Bc, jnp.float32(0.0)]).reshape(1, 6)

    encWT = jnp.zeros((D_IN, H), f32).at[:, :D_IN].set(
        enc_W.T * (SQRT_DM * A))
    encb = jnp.zeros((1, H), f32).at[0, :D_IN].set(enc_b * (SQRT_DM * A) + Bc)

    k_pad = jnp.maximum(gin_b2[0], 0.0)
    oadj = jnp.zeros((1, H), f32).at[0, D_IN].set(1.0 - k_pad)

    Wc = jnp.zeros((2 * H, 4 * H), f32)
    Wc = Wc.at[:D_IN, :].set(lstm_Wx.T)
    Wc = Wc.at[D_IN, :].set(lstm_bx + lstm_bh)
    Wc = Wc.at[H:, :].set(lstm_Wh.T)
    gate_scale = jnp.concatenate([
        jnp.full((H,), 0.5, f32), jnp.full((H,), 0.5, f32),
        jnp.ones((H,), f32), jnp.full((H,), 0.5, f32)]).reshape(1, 4 * H)
    Wc = Wc * gate_scale
    Wc = Wc.at[H:, :].mul(0.5)

    lenf = lengths.astype(f32).reshape(B, 1)
    inv = 0.5 / (lenf + 1.0)

    W1h = mlp_W1[:, :H].T
    W1s = mlp_W1[:, H:].T
    b1 = mlp_b1.reshape(1, H + D_STATIC)
    W2T = mlp_W2.T
    b2 = mlp_b2.reshape(1, N_CLASSES)

    full = lambda shape: pl.BlockSpec(shape, lambda b, t: tuple(0 for _ in shape))

    return pl.pallas_call(
        _fused_body,
        out_shape=jax.ShapeDtypeStruct((B, N_CLASSES), f32),
        grid=(NB, NT),
        in_specs=[
            pl.BlockSpec((TT, BB, D_IN), lambda b, t: (t, b, 0)),
            full((1, 6)),
            full((D_IN, H)),
            full((1, H)),
            full((2 * H, 4 * H)),
            full((1, H)),
            pl.BlockSpec((BB, 1), lambda b, t: (b, 0)),
            pl.BlockSpec((BB, 1), lambda b, t: (b, 0)),
            pl.BlockSpec((BB, D_STATIC), lambda b, t: (b, 0)),
            full((H, H + D_STATIC)),
            full((D_STATIC, H + D_STATIC)),
            full((1, H + D_STATIC)),
            full((H + D_STATIC, N_CLASSES)),
            full((1, N_CLASSES)),
        ],
        out_specs=pl.BlockSpec((BB, N_CLASSES), lambda b, t: (b, 0)),
        scratch_shapes=[
            pltpu.VMEM((BB, H), f32),
            pltpu.VMEM((BB, H), f32),
            pltpu.VMEM((BB, H), f32),
            pltpu.VMEM((8, H), f32),
            pltpu.VMEM((BB, H), f32),
            pltpu.VMEM((TT, BB, H), f32),
        ],
        compiler_params=pltpu.CompilerParams(
            dimension_semantics=("parallel", "arbitrary"),
        ),
        name="lstm_gin_fused",
    )(src, scal, encWT, encb, Wc, oadj, lenf, inv, static,
      W1h, W1s, b1, W2T, b2)
